# baseline (speedup 1.0000x reference)
import functools

import jax
import jax.numpy as jnp
import numpy as np
from jax import lax
from jax.experimental import pallas as pl
from jax.experimental.pallas import tpu as pltpu
from jax.experimental.pallas import tpu_sc as plsc

_N = 20000
_MAX_OUT = 100
_IOU_THRESH = 0.5
_L = 16
_NTILES = 16
_SH = 1264
_SHP = _SH + 16
_NP = _SH * _NTILES
_NCH = _SH // _L
_BIGI = np.int32(2**31 - 1)


def _nms_sc_body(x1h, y1h, x2h, y2h, sch, outh,
                 x1s, y1s, x2s, y2s, oss, css, pub, gath, outa, shared):
    cid = lax.axis_index("c")
    sid = lax.axis_index("s")
    base = sid * _SH
    pltpu.sync_copy(x1h.at[pl.ds(base, _SH)], x1s.at[pl.ds(0, _SH)])
    pltpu.sync_copy(y1h.at[pl.ds(base, _SH)], y1s.at[pl.ds(0, _SH)])
    pltpu.sync_copy(x2h.at[pl.ds(base, _SH)], x2s.at[pl.ds(0, _SH)])
    pltpu.sync_copy(y2h.at[pl.ds(base, _SH)], y2s.at[pl.ds(0, _SH)])
    pltpu.sync_copy(sch.at[pl.ds(base, _SH)], oss.at[pl.ds(0, _SH)])
    pltpu.sync_copy(sch.at[pl.ds(base, _SH)], css.at[pl.ds(0, _SH)])

    lane = lax.iota(jnp.int32, 16)
    neg_inf = np.float32(-np.inf)
    is_writer = (cid == 0) & (sid == 0)

    def splat_f(x):
        return jnp.full((16,), x, jnp.float32)

    def splat_i(x):
        return jnp.full((16,), x, jnp.int32)

    def shard_scan(c, carry):
        m, ids = carry
        off = c * _L
        v = css[pl.ds(off, _L)]
        ivec = splat_i(off) + lane
        upd = v > m
        return (jnp.where(upd, v, m), jnp.where(upd, ivec, ids))

    m0 = jnp.full((16,), neg_inf, jnp.float32)
    i0 = jnp.zeros((16,), jnp.int32)
    carry0 = lax.fori_loop(0, _NCH, shard_scan, (m0, i0))

    def iter_body(i, carry):
        m, ids = carry
        bs = m[0]
        bi = ids[0]
        for k in range(1, 16):
            v = m[k]
            ix = ids[k]
            better = (v > bs) | ((v == bs) & (ix < bi))
            bs = jnp.where(better, v, bs)
            bi = jnp.where(better, ix, bi)
        x1w = x1s[pl.ds(bi, 16)]
        y1w = y1s[pl.ds(bi, 16)]
        x2w = x2s[pl.ds(bi, 16)]
        y2w = y2s[pl.ds(bi, 16)]
        osw = oss[pl.ds(bi, 16)]
        pubv = jnp.where(lane == 0, bs,
               jnp.where(lane == 1, x1w[0],
               jnp.where(lane == 2, y1w[0],
               jnp.where(lane == 3, x2w[0],
               jnp.where(lane == 4, y2w[0],
               jnp.where(lane == 5, osw[0], np.float32(0.0)))))))
        pub[...] = pubv
        pltpu.sync_copy(pub, shared.at[pl.ds(sid * 16, 16)])
        plsc.subcore_barrier()
        pltpu.sync_copy(shared, gath.at[pl.ds(0, 256)])
        plsc.subcore_barrier()

        rec0 = gath[pl.ds(0, 16)]
        gs = rec0[0]
        gt = np.int32(0)
        for t in range(1, 16):
            rec = gath[pl.ds(t * 16, 16)]
            s = rec[0]
            better = s > gs
            gs = jnp.where(better, s, gs)
            gt = jnp.where(better, np.int32(t), gt)
        wrec = gath[pl.ds(gt * 16, 16)]
        x1b = splat_f(wrec[1])
        y1b = splat_f(wrec[2])
        x2b = splat_f(wrec[3])
        y2b = splat_f(wrec[4])
        a1 = (x2b - x1b) * (y2b - y1b)

        @pl.when(is_writer)
        def _():
            row = jnp.where(lane == 0, x1b,
                  jnp.where(lane == 1, y1b,
                  jnp.where(lane == 2, x2b,
                  jnp.where(lane == 3, y2b,
                  jnp.where(lane == 4, splat_f(wrec[5]), np.float32(0.0))))))
            outa[pl.ds(i * 16, 16)] = row

        def ch(c, carry2):
            m2, ids2 = carry2
            off = c * _L
            xv1 = x1s[pl.ds(off, _L)]
            yv1 = y1s[pl.ds(off, _L)]
            xv2 = x2s[pl.ds(off, _L)]
            yv2 = y2s[pl.ds(off, _L)]
            v = css[pl.ds(off, _L)]
            ix1 = jnp.maximum(x1b, xv1)
            iy1 = jnp.maximum(y1b, yv1)
            ix2 = jnp.minimum(x2b, xv2)
            iy2 = jnp.minimum(y2b, yv2)
            iw = jnp.maximum(ix2 - ix1, np.float32(0.0))
            ih = jnp.maximum(iy2 - iy1, np.float32(0.0))
            inter = iw * ih
            a2 = (xv2 - xv1) * (yv2 - yv1)
            union = (a1 + a2) - inter
            iou = inter / (union + np.float32(1e-9))
            v2 = jnp.where(iou > _IOU_THRESH, neg_inf, v)
            css[pl.ds(off, _L)] = v2
            ivec2 = splat_i(off) + lane
            upd = v2 > m2
            return (jnp.where(upd, v2, m2), jnp.where(upd, ivec2, ids2))

        return lax.fori_loop(0, _NCH, ch, (m0, i0))

    lax.fori_loop(0, _MAX_OUT, iter_body, carry0)

    @pl.when(is_writer)
    def _():
        pltpu.sync_copy(outa, outh)


@jax.jit
def kernel(boxes, scores):
    pad = _NP - _N
    x1 = jnp.pad(boxes[:, 0], (0, pad))
    y1 = jnp.pad(boxes[:, 1], (0, pad))
    x2 = jnp.pad(boxes[:, 2], (0, pad), constant_values=1.0)
    y2 = jnp.pad(boxes[:, 3], (0, pad), constant_values=1.0)
    sc = jnp.pad(scores, (0, pad), constant_values=-jnp.inf)
    mesh = plsc.VectorSubcoreMesh(core_axis_name="c", subcore_axis_name="s",
                                  num_cores=2, num_subcores=_NTILES)
    f = functools.partial(
        pl.kernel,
        mesh=mesh,
        compiler_params=pltpu.CompilerParams(needs_layout_passes=False),
        out_type=jax.ShapeDtypeStruct((_MAX_OUT * 16,), jnp.float32),
        scratch_types=[
            pltpu.VMEM((_SHP,), jnp.float32),
            pltpu.VMEM((_SHP,), jnp.float32),
            pltpu.VMEM((_SHP,), jnp.float32),
            pltpu.VMEM((_SHP,), jnp.float32),
            pltpu.VMEM((_SHP,), jnp.float32),
            pltpu.VMEM((_SHP,), jnp.float32),
            pltpu.VMEM((16,), jnp.float32),
            pltpu.VMEM((272,), jnp.float32),
            pltpu.VMEM((_MAX_OUT * 16,), jnp.float32),
            pltpu.VMEM_SHARED((256,), jnp.float32),
        ],
    )(_nms_sc_body)
    out = f(x1, y1, x2, y2, sc)
    return out.reshape(_MAX_OUT, 16)[:, :5]


# speedup vs baseline: 2.1227x; 2.1227x over previous
import functools

import jax
import jax.numpy as jnp
import numpy as np
from jax import lax
from jax.experimental import pallas as pl
from jax.experimental.pallas import tpu as pltpu
from jax.experimental.pallas import tpu_sc as plsc

_N = 20000
_MAX_OUT = 100
_IOU_THRESH = 0.5
_L = 16
_NTILES = 16
_SH = 1264
_SHP = _SH + 16
_NP = _SH * _NTILES
_NCH = _SH // _L
_BIGI = np.int32(2**31 - 1)


def _nms_sc_body(x1h, y1h, x2h, y2h, sch, outh,
                 x1s, y1s, x2s, y2s, oss, css, pub, gath, outa, shared):
    cid = lax.axis_index("c")
    sid = lax.axis_index("s")
    base = sid * _SH
    pltpu.sync_copy(x1h.at[pl.ds(base, _SH)], x1s.at[pl.ds(0, _SH)])
    pltpu.sync_copy(y1h.at[pl.ds(base, _SH)], y1s.at[pl.ds(0, _SH)])
    pltpu.sync_copy(x2h.at[pl.ds(base, _SH)], x2s.at[pl.ds(0, _SH)])
    pltpu.sync_copy(y2h.at[pl.ds(base, _SH)], y2s.at[pl.ds(0, _SH)])
    pltpu.sync_copy(sch.at[pl.ds(base, _SH)], oss.at[pl.ds(0, _SH)])
    pltpu.sync_copy(sch.at[pl.ds(base, _SH)], css.at[pl.ds(0, _SH)])

    lane = lax.iota(jnp.int32, 16)
    neg_inf = np.float32(-np.inf)
    css[pl.ds(_SH, _L)] = jnp.full((16,), neg_inf, jnp.float32)
    is_writer = (cid == 0) & (sid == 0)

    def splat_f(x):
        return jnp.full((16,), x, jnp.float32)

    def splat_i(x):
        return jnp.full((16,), x, jnp.int32)

    def shard_scan(c, carry):
        m, ids = carry
        off = c * _L
        v = css[pl.ds(off, _L)]
        ivec = splat_i(off) + lane
        upd = v > m
        return (jnp.where(upd, v, m), jnp.where(upd, ivec, ids))

    m0 = jnp.full((16,), neg_inf, jnp.float32)
    i0 = jnp.zeros((16,), jnp.int32)
    carry0 = lax.fori_loop(0, _NCH, shard_scan, (m0, i0))

    def iter_body(i, carry):
        m, ids = carry
        bs = m[0]
        bi = ids[0]
        for k in range(1, 16):
            v = m[k]
            ix = ids[k]
            better = (v > bs) | ((v == bs) & (ix < bi))
            bs = jnp.where(better, v, bs)
            bi = jnp.where(better, ix, bi)
        x1w = x1s[pl.ds(bi, 16)]
        y1w = y1s[pl.ds(bi, 16)]
        x2w = x2s[pl.ds(bi, 16)]
        y2w = y2s[pl.ds(bi, 16)]
        osw = oss[pl.ds(bi, 16)]
        pubv = jnp.where(lane == 0, bs,
               jnp.where(lane == 1, x1w[0],
               jnp.where(lane == 2, y1w[0],
               jnp.where(lane == 3, x2w[0],
               jnp.where(lane == 4, y2w[0],
               jnp.where(lane == 5, osw[0], np.float32(0.0)))))))
        pub[...] = pubv
        pltpu.sync_copy(pub, shared.at[pl.ds(sid * 16, 16)])
        plsc.subcore_barrier()
        pltpu.sync_copy(shared, gath.at[pl.ds(0, 256)])
        plsc.subcore_barrier()

        rec0 = gath[pl.ds(0, 16)]
        gs = rec0[0]
        gt = np.int32(0)
        for t in range(1, 16):
            rec = gath[pl.ds(t * 16, 16)]
            s = rec[0]
            better = s > gs
            gs = jnp.where(better, s, gs)
            gt = jnp.where(better, np.int32(t), gt)
        wrec = gath[pl.ds(gt * 16, 16)]
        x1b = splat_f(wrec[1])
        y1b = splat_f(wrec[2])
        x2b = splat_f(wrec[3])
        y2b = splat_f(wrec[4])
        a1 = (x2b - x1b) * (y2b - y1b)

        @pl.when(is_writer)
        def _():
            row = jnp.where(lane == 0, x1b,
                  jnp.where(lane == 1, y1b,
                  jnp.where(lane == 2, x2b,
                  jnp.where(lane == 3, y2b,
                  jnp.where(lane == 4, splat_f(wrec[5]), np.float32(0.0))))))
            outa[pl.ds(i * 16, 16)] = row

        @plsc.parallel_loop(0, _SHP, step=_L, unroll=8, carry=(m0, i0))
        def nxt(off, carry2):
            m2, ids2 = carry2
            xv1 = x1s[pl.ds(off, _L)]
            yv1 = y1s[pl.ds(off, _L)]
            xv2 = x2s[pl.ds(off, _L)]
            yv2 = y2s[pl.ds(off, _L)]
            v = css[pl.ds(off, _L)]
            ix1 = jnp.maximum(x1b, xv1)
            iy1 = jnp.maximum(y1b, yv1)
            ix2 = jnp.minimum(x2b, xv2)
            iy2 = jnp.minimum(y2b, yv2)
            iw = jnp.maximum(ix2 - ix1, np.float32(0.0))
            ih = jnp.maximum(iy2 - iy1, np.float32(0.0))
            inter = iw * ih
            a2 = (xv2 - xv1) * (yv2 - yv1)
            union = (a1 + a2) - inter
            iou = inter / (union + np.float32(1e-9))
            v2 = jnp.where(iou > _IOU_THRESH, neg_inf, v)
            css[pl.ds(off, _L)] = v2
            ivec2 = splat_i(off) + lane
            upd = v2 > m2
            return (jnp.where(upd, v2, m2), jnp.where(upd, ivec2, ids2))

        return nxt

    lax.fori_loop(0, _MAX_OUT, iter_body, carry0)

    @pl.when(is_writer)
    def _():
        pltpu.sync_copy(outa, outh)


@jax.jit
def kernel(boxes, scores):
    pad = _NP - _N
    x1 = jnp.pad(boxes[:, 0], (0, pad))
    y1 = jnp.pad(boxes[:, 1], (0, pad))
    x2 = jnp.pad(boxes[:, 2], (0, pad), constant_values=1.0)
    y2 = jnp.pad(boxes[:, 3], (0, pad), constant_values=1.0)
    sc = jnp.pad(scores, (0, pad), constant_values=-jnp.inf)
    mesh = plsc.VectorSubcoreMesh(core_axis_name="c", subcore_axis_name="s",
                                  num_cores=2, num_subcores=_NTILES)
    f = functools.partial(
        pl.kernel,
        mesh=mesh,
        compiler_params=pltpu.CompilerParams(needs_layout_passes=False),
        out_type=jax.ShapeDtypeStruct((_MAX_OUT * 16,), jnp.float32),
        scratch_types=[
            pltpu.VMEM((_SHP,), jnp.float32),
            pltpu.VMEM((_SHP,), jnp.float32),
            pltpu.VMEM((_SHP,), jnp.float32),
            pltpu.VMEM((_SHP,), jnp.float32),
            pltpu.VMEM((_SHP,), jnp.float32),
            pltpu.VMEM((_SHP,), jnp.float32),
            pltpu.VMEM((16,), jnp.float32),
            pltpu.VMEM((272,), jnp.float32),
            pltpu.VMEM((_MAX_OUT * 16,), jnp.float32),
            pltpu.VMEM_SHARED((256,), jnp.float32),
        ],
    )(_nms_sc_body)
    out = f(x1, y1, x2, y2, sc)
    return out.reshape(_MAX_OUT, 16)[:, :5]


# speedup vs baseline: 2.3082x; 1.0874x over previous
import functools

import jax
import jax.numpy as jnp
import numpy as np
from jax import lax
from jax.experimental import pallas as pl
from jax.experimental.pallas import tpu as pltpu
from jax.experimental.pallas import tpu_sc as plsc

_N = 20000
_MAX_OUT = 100
_IOU_THRESH = 0.5
_L = 16
_NTILES = 16
_SH = 1264
_SHP = _SH + 16
_NP = _SH * _NTILES
_NCH = _SH // _L
_BIGI = np.int32(2**31 - 1)


def _nms_sc_body(x1h, y1h, x2h, y2h, sch, outh,
                 x1s, y1s, x2s, y2s, oss, css, pub, gath, outa, shared):
    cid = lax.axis_index("c")
    sid = lax.axis_index("s")
    base = sid * _SH
    pltpu.sync_copy(x1h.at[pl.ds(base, _SH)], x1s.at[pl.ds(0, _SH)])
    pltpu.sync_copy(y1h.at[pl.ds(base, _SH)], y1s.at[pl.ds(0, _SH)])
    pltpu.sync_copy(x2h.at[pl.ds(base, _SH)], x2s.at[pl.ds(0, _SH)])
    pltpu.sync_copy(y2h.at[pl.ds(base, _SH)], y2s.at[pl.ds(0, _SH)])
    pltpu.sync_copy(sch.at[pl.ds(base, _SH)], oss.at[pl.ds(0, _SH)])
    pltpu.sync_copy(sch.at[pl.ds(base, _SH)], css.at[pl.ds(0, _SH)])

    lane = lax.iota(jnp.int32, 16)
    neg_inf = np.float32(-np.inf)
    css[pl.ds(_SH, _L)] = jnp.full((16,), neg_inf, jnp.float32)
    is_writer = (cid == 0) & (sid == 0)

    def splat_f(x):
        return jnp.full((16,), x, jnp.float32)

    def splat_i(x):
        return jnp.full((16,), x, jnp.int32)

    def shard_scan(c, carry):
        m, ids = carry
        off = c * _L
        v = css[pl.ds(off, _L)]
        ivec = splat_i(off) + lane
        upd = v > m
        return (jnp.where(upd, v, m), jnp.where(upd, ivec, ids))

    m0 = jnp.full((16,), neg_inf, jnp.float32)
    i0 = jnp.zeros((16,), jnp.int32)
    carry0 = lax.fori_loop(0, _NCH, shard_scan, (m0, i0))

    perms = [jnp.bitwise_and(lane + s, 15) for s in (8, 4, 2, 1)]

    def iter_body(i, carry):
        m, ids = carry
        for p in perms:
            mp = jnp.take_along_axis(m, p, axis=0)
            ip = jnp.take_along_axis(ids, p, axis=0)
            better = (mp > m) | ((mp == m) & (ip < ids))
            m = jnp.where(better, mp, m)
            ids = jnp.where(better, ip, ids)
        bs = m[0]
        bi = ids[0]
        x1w = x1s[pl.ds(bi, 16)]
        y1w = y1s[pl.ds(bi, 16)]
        x2w = x2s[pl.ds(bi, 16)]
        y2w = y2s[pl.ds(bi, 16)]
        osw = oss[pl.ds(bi, 16)]
        pubv = jnp.where(lane == 0, bs,
               jnp.where(lane == 1, x1w[0],
               jnp.where(lane == 2, y1w[0],
               jnp.where(lane == 3, x2w[0],
               jnp.where(lane == 4, y2w[0],
               jnp.where(lane == 5, osw[0], np.float32(0.0)))))))
        pub[...] = pubv
        buf = jnp.bitwise_and(i, 1) * 256
        pltpu.sync_copy(pub, shared.at[pl.ds(buf + sid * 16, 16)])
        plsc.subcore_barrier()
        pltpu.sync_copy(shared.at[pl.ds(buf, 256)], gath.at[pl.ds(0, 256)])

        sv = plsc.load_gather(gath, [lane * 16])
        tv = lane
        for p in perms:
            sp = jnp.take_along_axis(sv, p, axis=0)
            tp = jnp.take_along_axis(tv, p, axis=0)
            better = (sp > sv) | ((sp == sv) & (tp < tv))
            sv = jnp.where(better, sp, sv)
            tv = jnp.where(better, tp, tv)
        gt = tv[0]
        wrec = gath[pl.ds(gt * 16, 16)]
        x1b = splat_f(wrec[1])
        y1b = splat_f(wrec[2])
        x2b = splat_f(wrec[3])
        y2b = splat_f(wrec[4])
        a1 = (x2b - x1b) * (y2b - y1b)

        @pl.when(is_writer)
        def _():
            row = jnp.where(lane == 0, x1b,
                  jnp.where(lane == 1, y1b,
                  jnp.where(lane == 2, x2b,
                  jnp.where(lane == 3, y2b,
                  jnp.where(lane == 4, splat_f(wrec[5]), np.float32(0.0))))))
            outa[pl.ds(i * 16, 16)] = row

        @plsc.parallel_loop(0, _SHP, step=_L, unroll=8, carry=(m0, i0))
        def nxt(off, carry2):
            m2, ids2 = carry2
            xv1 = x1s[pl.ds(off, _L)]
            yv1 = y1s[pl.ds(off, _L)]
            xv2 = x2s[pl.ds(off, _L)]
            yv2 = y2s[pl.ds(off, _L)]
            v = css[pl.ds(off, _L)]
            ix1 = jnp.maximum(x1b, xv1)
            iy1 = jnp.maximum(y1b, yv1)
            ix2 = jnp.minimum(x2b, xv2)
            iy2 = jnp.minimum(y2b, yv2)
            iw = jnp.maximum(ix2 - ix1, np.float32(0.0))
            ih = jnp.maximum(iy2 - iy1, np.float32(0.0))
            inter = iw * ih
            a2 = (xv2 - xv1) * (yv2 - yv1)
            union = (a1 + a2) - inter
            iou = inter / (union + np.float32(1e-9))
            v2 = jnp.where(iou > _IOU_THRESH, neg_inf, v)
            css[pl.ds(off, _L)] = v2
            ivec2 = splat_i(off) + lane
            upd = v2 > m2
            return (jnp.where(upd, v2, m2), jnp.where(upd, ivec2, ids2))

        return nxt

    lax.fori_loop(0, _MAX_OUT, iter_body, carry0)

    @pl.when(is_writer)
    def _():
        pltpu.sync_copy(outa, outh)


@jax.jit
def kernel(boxes, scores):
    pad = _NP - _N
    x1 = jnp.pad(boxes[:, 0], (0, pad))
    y1 = jnp.pad(boxes[:, 1], (0, pad))
    x2 = jnp.pad(boxes[:, 2], (0, pad), constant_values=1.0)
    y2 = jnp.pad(boxes[:, 3], (0, pad), constant_values=1.0)
    sc = jnp.pad(scores, (0, pad), constant_values=-jnp.inf)
    mesh = plsc.VectorSubcoreMesh(core_axis_name="c", subcore_axis_name="s",
                                  num_cores=2, num_subcores=_NTILES)
    f = functools.partial(
        pl.kernel,
        mesh=mesh,
        compiler_params=pltpu.CompilerParams(needs_layout_passes=False),
        out_type=jax.ShapeDtypeStruct((_MAX_OUT * 16,), jnp.float32),
        scratch_types=[
            pltpu.VMEM((_SHP,), jnp.float32),
            pltpu.VMEM((_SHP,), jnp.float32),
            pltpu.VMEM((_SHP,), jnp.float32),
            pltpu.VMEM((_SHP,), jnp.float32),
            pltpu.VMEM((_SHP,), jnp.float32),
            pltpu.VMEM((_SHP,), jnp.float32),
            pltpu.VMEM((16,), jnp.float32),
            pltpu.VMEM((272,), jnp.float32),
            pltpu.VMEM((_MAX_OUT * 16,), jnp.float32),
            pltpu.VMEM_SHARED((512,), jnp.float32),
        ],
    )(_nms_sc_body)
    out = f(x1, y1, x2, y2, sc)
    return out.reshape(_MAX_OUT, 16)[:, :5]


# speedup vs baseline: 2.4211x; 1.0489x over previous
import functools

import jax
import jax.numpy as jnp
import numpy as np
from jax import lax
from jax.experimental import pallas as pl
from jax.experimental.pallas import tpu as pltpu
from jax.experimental.pallas import tpu_sc as plsc

_N = 20000
_MAX_OUT = 100
_IOU_THRESH = 0.5
_L = 16
_NTILES = 16
_SH = 1264
_SHP = _SH + 16
_NP = _SH * _NTILES
_NCH = _SH // _L
_BIGI = np.int32(2**31 - 1)


def _nms_sc_body(x1h, y1h, x2h, y2h, sch, outh,
                 x1s, y1s, x2s, y2s, oss, css, a2s, pub, gath, outa, shared):
    cid = lax.axis_index("c")
    sid = lax.axis_index("s")
    base = sid * _SH
    pltpu.sync_copy(x1h.at[pl.ds(base, _SH)], x1s.at[pl.ds(0, _SH)])
    pltpu.sync_copy(y1h.at[pl.ds(base, _SH)], y1s.at[pl.ds(0, _SH)])
    pltpu.sync_copy(x2h.at[pl.ds(base, _SH)], x2s.at[pl.ds(0, _SH)])
    pltpu.sync_copy(y2h.at[pl.ds(base, _SH)], y2s.at[pl.ds(0, _SH)])
    pltpu.sync_copy(sch.at[pl.ds(base, _SH)], oss.at[pl.ds(0, _SH)])
    pltpu.sync_copy(sch.at[pl.ds(base, _SH)], css.at[pl.ds(0, _SH)])

    lane = lax.iota(jnp.int32, 16)
    neg_inf = np.float32(-np.inf)
    css[pl.ds(_SH, _L)] = jnp.full((16,), neg_inf, jnp.float32)
    is_writer = (cid == 0) & (sid == 0)

    def splat_f(x):
        return jnp.full((16,), x, jnp.float32)

    def splat_i(x):
        return jnp.full((16,), x, jnp.int32)

    def shard_scan(c, carry):
        m, ids = carry
        off = c * _L
        v = css[pl.ds(off, _L)]
        ivec = splat_i(off) + lane
        upd = v > m
        return (jnp.where(upd, v, m), jnp.where(upd, ivec, ids))

    m0 = jnp.full((16,), neg_inf, jnp.float32)
    i0 = jnp.zeros((16,), jnp.int32)

    @plsc.parallel_loop(0, _SHP, step=_L, unroll=8)
    def _a2_init(off):
        a2s[pl.ds(off, _L)] = ((x2s[pl.ds(off, _L)] - x1s[pl.ds(off, _L)]) *
                               (y2s[pl.ds(off, _L)] - y1s[pl.ds(off, _L)]))

    carry0 = lax.fori_loop(0, _NCH, shard_scan, (m0, i0))

    perms = [jnp.bitwise_and(lane + s, 15) for s in (8, 4, 2, 1)]

    def iter_body(i, carry):
        m, ids = carry
        for p in perms:
            mp = jnp.take_along_axis(m, p, axis=0)
            ip = jnp.take_along_axis(ids, p, axis=0)
            better = (mp > m) | ((mp == m) & (ip < ids))
            m = jnp.where(better, mp, m)
            ids = jnp.where(better, ip, ids)
        bs = m[0]
        bi = ids[0]
        x1w = x1s[pl.ds(bi, 16)]
        y1w = y1s[pl.ds(bi, 16)]
        x2w = x2s[pl.ds(bi, 16)]
        y2w = y2s[pl.ds(bi, 16)]
        osw = oss[pl.ds(bi, 16)]
        pubv = jnp.where(lane == 0, bs,
               jnp.where(lane == 1, x1w[0],
               jnp.where(lane == 2, y1w[0],
               jnp.where(lane == 3, x2w[0],
               jnp.where(lane == 4, y2w[0],
               jnp.where(lane == 5, osw[0], np.float32(0.0)))))))
        pub[...] = pubv
        buf = jnp.bitwise_and(i, 1) * 256
        pltpu.sync_copy(pub, shared.at[pl.ds(buf + sid * 16, 16)])
        plsc.subcore_barrier()
        pltpu.sync_copy(shared.at[pl.ds(buf, 256)], gath.at[pl.ds(0, 256)])

        sv = plsc.load_gather(gath, [lane * 16])
        tv = lane
        for p in perms:
            sp = jnp.take_along_axis(sv, p, axis=0)
            tp = jnp.take_along_axis(tv, p, axis=0)
            better = (sp > sv) | ((sp == sv) & (tp < tv))
            sv = jnp.where(better, sp, sv)
            tv = jnp.where(better, tp, tv)
        gt = tv[0]
        wrec = gath[pl.ds(gt * 16, 16)]
        x1b = splat_f(wrec[1])
        y1b = splat_f(wrec[2])
        x2b = splat_f(wrec[3])
        y2b = splat_f(wrec[4])
        a1 = (x2b - x1b) * (y2b - y1b)

        @pl.when(is_writer)
        def _():
            row = jnp.where(lane == 0, x1b,
                  jnp.where(lane == 1, y1b,
                  jnp.where(lane == 2, x2b,
                  jnp.where(lane == 3, y2b,
                  jnp.where(lane == 4, splat_f(wrec[5]), np.float32(0.0))))))
            outa[pl.ds(i * 16, 16)] = row

        @plsc.parallel_loop(0, _SHP, step=_L, unroll=8, carry=(m0, i0))
        def nxt(off, carry2):
            m2, ids2 = carry2
            xv1 = x1s[pl.ds(off, _L)]
            yv1 = y1s[pl.ds(off, _L)]
            xv2 = x2s[pl.ds(off, _L)]
            yv2 = y2s[pl.ds(off, _L)]
            v = css[pl.ds(off, _L)]
            ix1 = jnp.maximum(x1b, xv1)
            iy1 = jnp.maximum(y1b, yv1)
            ix2 = jnp.minimum(x2b, xv2)
            iy2 = jnp.minimum(y2b, yv2)
            iw = jnp.maximum(ix2 - ix1, np.float32(0.0))
            ih = jnp.maximum(iy2 - iy1, np.float32(0.0))
            inter = iw * ih
            a2 = a2s[pl.ds(off, _L)]
            union = (a1 + a2) - inter
            iou = inter / (union + np.float32(1e-9))
            v2 = jnp.where(iou > _IOU_THRESH, neg_inf, v)
            css[pl.ds(off, _L)] = v2
            ivec2 = splat_i(off) + lane
            upd = v2 > m2
            return (jnp.where(upd, v2, m2), jnp.where(upd, ivec2, ids2))

        return nxt

    lax.fori_loop(0, _MAX_OUT, iter_body, carry0)

    @pl.when(is_writer)
    def _():
        pltpu.sync_copy(outa, outh)


@jax.jit
def kernel(boxes, scores):
    pad = _NP - _N
    x1 = jnp.pad(boxes[:, 0], (0, pad))
    y1 = jnp.pad(boxes[:, 1], (0, pad))
    x2 = jnp.pad(boxes[:, 2], (0, pad), constant_values=1.0)
    y2 = jnp.pad(boxes[:, 3], (0, pad), constant_values=1.0)
    sc = jnp.pad(scores, (0, pad), constant_values=-jnp.inf)
    mesh = plsc.VectorSubcoreMesh(core_axis_name="c", subcore_axis_name="s",
                                  num_cores=2, num_subcores=_NTILES)
    f = functools.partial(
        pl.kernel,
        mesh=mesh,
        compiler_params=pltpu.CompilerParams(needs_layout_passes=False),
        out_type=jax.ShapeDtypeStruct((_MAX_OUT * 16,), jnp.float32),
        scratch_types=[
            pltpu.VMEM((_SHP,), jnp.float32),
            pltpu.VMEM((_SHP,), jnp.float32),
            pltpu.VMEM((_SHP,), jnp.float32),
            pltpu.VMEM((_SHP,), jnp.float32),
            pltpu.VMEM((_SHP,), jnp.float32),
            pltpu.VMEM((_SHP,), jnp.float32),
            pltpu.VMEM((_SHP,), jnp.float32),
            pltpu.VMEM((16,), jnp.float32),
            pltpu.VMEM((272,), jnp.float32),
            pltpu.VMEM((_MAX_OUT * 16,), jnp.float32),
            pltpu.VMEM_SHARED((512,), jnp.float32),
        ],
    )(_nms_sc_body)
    out = f(x1, y1, x2, y2, sc)
    return out.reshape(_MAX_OUT, 16)[:, :5]


# speedup vs baseline: 2.4219x; 1.0003x over previous
import functools

import jax
import jax.numpy as jnp
import numpy as np
from jax import lax
from jax.experimental import pallas as pl
from jax.experimental.pallas import tpu as pltpu
from jax.experimental.pallas import tpu_sc as plsc

_N = 20000
_MAX_OUT = 100
_IOU_THRESH = 0.5
_L = 16
_NTILES = 16
_SH = 1264
_SHP = _SH + 16
_NP = _SH * _NTILES
_NCH = _SH // _L
_BIGI = np.int32(2**31 - 1)


def _nms_sc_body(x1h, y1h, x2h, y2h, sch, outh,
                 x1s, y1s, x2s, y2s, oss, css, a2s, pub, gath, outa, shared):
    cid = lax.axis_index("c")
    sid = lax.axis_index("s")
    base = sid * _SH
    pltpu.sync_copy(x1h.at[pl.ds(base, _SH)], x1s.at[pl.ds(0, _SH)])
    pltpu.sync_copy(y1h.at[pl.ds(base, _SH)], y1s.at[pl.ds(0, _SH)])
    pltpu.sync_copy(x2h.at[pl.ds(base, _SH)], x2s.at[pl.ds(0, _SH)])
    pltpu.sync_copy(y2h.at[pl.ds(base, _SH)], y2s.at[pl.ds(0, _SH)])
    pltpu.sync_copy(sch.at[pl.ds(base, _SH)], oss.at[pl.ds(0, _SH)])
    pltpu.sync_copy(sch.at[pl.ds(base, _SH)], css.at[pl.ds(0, _SH)])

    lane = lax.iota(jnp.int32, 16)
    neg_inf = np.float32(-np.inf)
    css[pl.ds(_SH, _L)] = jnp.full((16,), neg_inf, jnp.float32)
    is_writer = (cid == 0) & (sid == 0)

    def splat_f(x):
        return jnp.full((16,), x, jnp.float32)

    def splat_i(x):
        return jnp.full((16,), x, jnp.int32)

    m0 = jnp.full((16,), neg_inf, jnp.float32)
    i0 = jnp.zeros((16,), jnp.int32)

    @plsc.parallel_loop(0, _SHP, step=_L, unroll=8)
    def _a2_init(off):
        a2s[pl.ds(off, _L)] = ((x2s[pl.ds(off, _L)] - x1s[pl.ds(off, _L)]) *
                               (y2s[pl.ds(off, _L)] - y1s[pl.ds(off, _L)]))

    @plsc.parallel_loop(0, _SHP, step=_L, unroll=8, carry=(m0, i0))
    def carry0(off, carry):
        m, ids = carry
        v = css[pl.ds(off, _L)]
        ivec = splat_i(off) + lane
        upd = v > m
        return (jnp.where(upd, v, m), jnp.where(upd, ivec, ids))

    perms = [jnp.bitwise_and(lane + s, 15) for s in (8, 4, 2, 1)]

    def iter_body(i, carry):
        m, ids = carry
        for p in perms:
            mp = jnp.take_along_axis(m, p, axis=0)
            ip = jnp.take_along_axis(ids, p, axis=0)
            better = (mp > m) | ((mp == m) & (ip < ids))
            m = jnp.where(better, mp, m)
            ids = jnp.where(better, ip, ids)
        bs = m[0]
        bi = ids[0]
        x1w = x1s[pl.ds(bi, 16)]
        y1w = y1s[pl.ds(bi, 16)]
        x2w = x2s[pl.ds(bi, 16)]
        y2w = y2s[pl.ds(bi, 16)]
        osw = oss[pl.ds(bi, 16)]
        pubv = jnp.where(lane == 0, bs,
               jnp.where(lane == 1, x1w[0],
               jnp.where(lane == 2, y1w[0],
               jnp.where(lane == 3, x2w[0],
               jnp.where(lane == 4, y2w[0],
               jnp.where(lane == 5, osw[0], np.float32(0.0)))))))
        pub[...] = pubv
        buf = jnp.bitwise_and(i, 1) * 256
        pltpu.sync_copy(pub, shared.at[pl.ds(buf + sid * 16, 16)])
        plsc.subcore_barrier()
        pltpu.sync_copy(shared.at[pl.ds(buf, 256)], gath.at[pl.ds(0, 256)])

        sv = plsc.load_gather(gath, [lane * 16])
        tv = lane
        for p in perms:
            sp = jnp.take_along_axis(sv, p, axis=0)
            tp = jnp.take_along_axis(tv, p, axis=0)
            better = (sp > sv) | ((sp == sv) & (tp < tv))
            sv = jnp.where(better, sp, sv)
            tv = jnp.where(better, tp, tv)
        gt = tv[0]
        wrec = gath[pl.ds(gt * 16, 16)]
        x1b = splat_f(wrec[1])
        y1b = splat_f(wrec[2])
        x2b = splat_f(wrec[3])
        y2b = splat_f(wrec[4])
        a1 = (x2b - x1b) * (y2b - y1b)

        @pl.when(is_writer)
        def _():
            row = jnp.where(lane == 0, x1b,
                  jnp.where(lane == 1, y1b,
                  jnp.where(lane == 2, x2b,
                  jnp.where(lane == 3, y2b,
                  jnp.where(lane == 4, splat_f(wrec[5]), np.float32(0.0))))))
            outa[pl.ds(i * 16, 16)] = row

        @plsc.parallel_loop(0, _SHP, step=_L, unroll=8, carry=(m0, i0))
        def nxt(off, carry2):
            m2, ids2 = carry2
            xv1 = x1s[pl.ds(off, _L)]
            yv1 = y1s[pl.ds(off, _L)]
            xv2 = x2s[pl.ds(off, _L)]
            yv2 = y2s[pl.ds(off, _L)]
            v = css[pl.ds(off, _L)]
            ix1 = jnp.maximum(x1b, xv1)
            iy1 = jnp.maximum(y1b, yv1)
            ix2 = jnp.minimum(x2b, xv2)
            iy2 = jnp.minimum(y2b, yv2)
            iw = jnp.maximum(ix2 - ix1, np.float32(0.0))
            ih = jnp.maximum(iy2 - iy1, np.float32(0.0))
            inter = iw * ih
            a2 = a2s[pl.ds(off, _L)]
            union = (a1 + a2) - inter
            iou = inter / (union + np.float32(1e-9))
            v2 = jnp.where(iou > _IOU_THRESH, neg_inf, v)
            css[pl.ds(off, _L)] = v2
            ivec2 = splat_i(off) + lane
            upd = v2 > m2
            return (jnp.where(upd, v2, m2), jnp.where(upd, ivec2, ids2))

        return nxt

    lax.fori_loop(0, _MAX_OUT, iter_body, carry0)

    @pl.when(is_writer)
    def _():
        pltpu.sync_copy(outa, outh)


@jax.jit
def kernel(boxes, scores):
    pad = _NP - _N
    x1 = jnp.pad(boxes[:, 0], (0, pad))
    y1 = jnp.pad(boxes[:, 1], (0, pad))
    x2 = jnp.pad(boxes[:, 2], (0, pad), constant_values=1.0)
    y2 = jnp.pad(boxes[:, 3], (0, pad), constant_values=1.0)
    sc = jnp.pad(scores, (0, pad), constant_values=-jnp.inf)
    mesh = plsc.VectorSubcoreMesh(core_axis_name="c", subcore_axis_name="s",
                                  num_cores=2, num_subcores=_NTILES)
    f = functools.partial(
        pl.kernel,
        mesh=mesh,
        compiler_params=pltpu.CompilerParams(needs_layout_passes=False),
        out_type=jax.ShapeDtypeStruct((_MAX_OUT * 16,), jnp.float32),
        scratch_types=[
            pltpu.VMEM((_SHP,), jnp.float32),
            pltpu.VMEM((_SHP,), jnp.float32),
            pltpu.VMEM((_SHP,), jnp.float32),
            pltpu.VMEM((_SHP,), jnp.float32),
            pltpu.VMEM((_SHP,), jnp.float32),
            pltpu.VMEM((_SHP,), jnp.float32),
            pltpu.VMEM((_SHP,), jnp.float32),
            pltpu.VMEM((16,), jnp.float32),
            pltpu.VMEM((272,), jnp.float32),
            pltpu.VMEM((_MAX_OUT * 16,), jnp.float32),
            pltpu.VMEM_SHARED((512,), jnp.float32),
        ],
    )(_nms_sc_body)
    out = f(x1, y1, x2, y2, sc)
    return out.reshape(_MAX_OUT, 16)[:, :5]


# speedup vs baseline: 2.4314x; 1.0040x over previous
import functools

import jax
import jax.numpy as jnp
import numpy as np
from jax import lax
from jax.experimental import pallas as pl
from jax.experimental.pallas import tpu as pltpu
from jax.experimental.pallas import tpu_sc as plsc

_N = 20000
_MAX_OUT = 100
_IOU_THRESH = 0.5
_L = 16
_NTILES = 16
_SH = 1264
_SHP = _SH + 16
_NP = _SH * _NTILES


def _nms_sc_body(x1h, y1h, x2h, y2h, sch, outh,
                 x1s, y1s, x2s, y2s, oss, css, a2s, pub, gath, outa, shared):
    cid = lax.axis_index("c")
    sid = lax.axis_index("s")
    base = sid * _SH
    pltpu.sync_copy(x1h.at[pl.ds(base, _SH)], x1s.at[pl.ds(0, _SH)])
    pltpu.sync_copy(y1h.at[pl.ds(base, _SH)], y1s.at[pl.ds(0, _SH)])
    pltpu.sync_copy(x2h.at[pl.ds(base, _SH)], x2s.at[pl.ds(0, _SH)])
    pltpu.sync_copy(y2h.at[pl.ds(base, _SH)], y2s.at[pl.ds(0, _SH)])
    pltpu.sync_copy(sch.at[pl.ds(base, _SH)], oss.at[pl.ds(0, _SH)])
    pltpu.sync_copy(sch.at[pl.ds(base, _SH)], css.at[pl.ds(0, _SH)])

    lane = lax.iota(jnp.int32, 16)
    neg_inf = np.float32(-np.inf)
    css[pl.ds(_SH, _L)] = jnp.full((16,), neg_inf, jnp.float32)
    is_writer = (cid == 0) & (sid == 0)

    def splat_f(x):
        return jnp.full((16,), x, jnp.float32)

    def splat_i(x):
        return jnp.full((16,), x, jnp.int32)

    m0 = jnp.full((16,), neg_inf, jnp.float32)
    i0 = jnp.zeros((16,), jnp.int32)

    @plsc.parallel_loop(0, _SHP, step=_L, unroll=8)
    def _a2_init(off):
        a2s[pl.ds(off, _L)] = ((x2s[pl.ds(off, _L)] - x1s[pl.ds(off, _L)]) *
                               (y2s[pl.ds(off, _L)] - y1s[pl.ds(off, _L)]))

    @plsc.parallel_loop(0, _SHP, step=_L, unroll=8, carry=(m0, i0))
    def carry0(off, carry):
        m, ids = carry
        v = css[pl.ds(off, _L)]
        ivec = splat_i(off) + lane
        upd = v > m
        return (jnp.where(upd, v, m), jnp.where(upd, ivec, ids))

    perms = [jnp.bitwise_and(lane + s, 15) for s in (8, 4, 2, 1)]

    def iter_body(i, carry):
        m, ids = carry
        for p in perms:
            mp = jnp.take_along_axis(m, p, axis=0)
            ip = jnp.take_along_axis(ids, p, axis=0)
            better = (mp > m) | ((mp == m) & (ip < ids))
            m = jnp.where(better, mp, m)
            ids = jnp.where(better, ip, ids)
        bs = m[0]
        bi = ids[0]
        x1w = x1s[pl.ds(bi, 16)]
        y1w = y1s[pl.ds(bi, 16)]
        x2w = x2s[pl.ds(bi, 16)]
        y2w = y2s[pl.ds(bi, 16)]
        osw = oss[pl.ds(bi, 16)]
        pubv = jnp.where(lane == 0, bs,
               jnp.where(lane == 1, x1w[0],
               jnp.where(lane == 2, y1w[0],
               jnp.where(lane == 3, x2w[0],
               jnp.where(lane == 4, y2w[0],
               jnp.where(lane == 5, osw[0], np.float32(0.0)))))))
        pub[...] = pubv
        buf = jnp.bitwise_and(i, 1) * 256
        pltpu.sync_copy(pub, shared.at[pl.ds(buf + sid * 16, 16)])
        plsc.subcore_barrier()
        pltpu.sync_copy(shared.at[pl.ds(buf, 256)], gath.at[pl.ds(0, 256)])

        sv = plsc.load_gather(gath, [lane * 16])
        tv = lane
        for p in perms:
            sp = jnp.take_along_axis(sv, p, axis=0)
            tp = jnp.take_along_axis(tv, p, axis=0)
            better = (sp > sv) | ((sp == sv) & (tp < tv))
            sv = jnp.where(better, sp, sv)
            tv = jnp.where(better, tp, tv)
        gt = tv[0]
        wrec = gath[pl.ds(gt * 16, 16)]
        x1b = splat_f(wrec[1])
        y1b = splat_f(wrec[2])
        x2b = splat_f(wrec[3])
        y2b = splat_f(wrec[4])
        a1 = (x2b - x1b) * (y2b - y1b)

        @pl.when(is_writer)
        def _():
            row = jnp.where(lane == 0, x1b,
                  jnp.where(lane == 1, y1b,
                  jnp.where(lane == 2, x2b,
                  jnp.where(lane == 3, y2b,
                  jnp.where(lane == 4, splat_f(wrec[5]), np.float32(0.0))))))
            outa[pl.ds(i * 16, 16)] = row

        @plsc.parallel_loop(0, _SHP, step=_L, unroll=8, carry=(m0, i0))
        def nxt(off, carry2):
            m2, ids2 = carry2
            xv1 = x1s[pl.ds(off, _L)]
            yv1 = y1s[pl.ds(off, _L)]
            xv2 = x2s[pl.ds(off, _L)]
            yv2 = y2s[pl.ds(off, _L)]
            v = css[pl.ds(off, _L)]
            ix1 = jnp.maximum(x1b, xv1)
            iy1 = jnp.maximum(y1b, yv1)
            ix2 = jnp.minimum(x2b, xv2)
            iy2 = jnp.minimum(y2b, yv2)
            iw = jnp.maximum(ix2 - ix1, np.float32(0.0))
            ih = jnp.maximum(iy2 - iy1, np.float32(0.0))
            inter = iw * ih
            a2 = a2s[pl.ds(off, _L)]
            union = (a1 + a2) - inter
            iou = inter / (union + np.float32(1e-9))
            v2 = jnp.where(iou > _IOU_THRESH, neg_inf, v)
            css[pl.ds(off, _L)] = v2
            ivec2 = splat_i(off) + lane
            upd = v2 > m2
            return (jnp.where(upd, v2, m2), jnp.where(upd, ivec2, ids2))

        return nxt

    lax.fori_loop(0, _MAX_OUT, iter_body, carry0)

    @pl.when(is_writer)
    def _():
        pltpu.sync_copy(outa, outh)


@jax.jit
def kernel(boxes, scores):
    pad = _NP - _N
    x1 = jnp.pad(boxes[:, 0], (0, pad))
    y1 = jnp.pad(boxes[:, 1], (0, pad))
    x2 = jnp.pad(boxes[:, 2], (0, pad), constant_values=1.0)
    y2 = jnp.pad(boxes[:, 3], (0, pad), constant_values=1.0)
    sc = jnp.pad(scores, (0, pad), constant_values=-jnp.inf)
    mesh = plsc.VectorSubcoreMesh(core_axis_name="c", subcore_axis_name="s",
                                  num_cores=2, num_subcores=_NTILES)
    f = functools.partial(
        pl.kernel,
        mesh=mesh,
        compiler_params=pltpu.CompilerParams(needs_layout_passes=False),
        out_type=jax.ShapeDtypeStruct((_MAX_OUT * 16,), jnp.float32),
        scratch_types=[
            pltpu.VMEM((_SHP,), jnp.float32),
            pltpu.VMEM((_SHP,), jnp.float32),
            pltpu.VMEM((_SHP,), jnp.float32),
            pltpu.VMEM((_SHP,), jnp.float32),
            pltpu.VMEM((_SHP,), jnp.float32),
            pltpu.VMEM((_SHP,), jnp.float32),
            pltpu.VMEM((_SHP,), jnp.float32),
            pltpu.VMEM((16,), jnp.float32),
            pltpu.VMEM((272,), jnp.float32),
            pltpu.VMEM((_MAX_OUT * 16,), jnp.float32),
            pltpu.VMEM_SHARED((512,), jnp.float32),
        ],
    )(_nms_sc_body)
    out = f(x1, y1, x2, y2, sc)
    return out.reshape(_MAX_OUT, 16)[:, :5]
